# baseline (speedup 1.0000x reference)
import functools

import jax
import jax.numpy as jnp
from jax import lax
from jax.experimental import pallas as pl
from jax.experimental.pallas import tpu as pltpu
from jax.experimental.pallas import tpu_sc as plsc

_B = 1024
_T = 50
_D = 32
_DEC = 20

_info = plsc.get_sparse_core_info()
_NC = _info.num_cores
_NS = _info.num_subcores
_NW = _NC * _NS
_NB = _B // _NW


@functools.partial(
    pl.kernel,
    mesh=plsc.VectorSubcoreMesh(core_axis_name="c", subcore_axis_name="s"),
    out_type=jax.ShapeDtypeStruct((_B, _DEC, _T, _D), jnp.float32),
    compiler_params=pltpu.CompilerParams(use_tc_tiling_on_sc=False),
    scratch_types=[
        pltpu.VMEM((_NB, _T), jnp.int32),
        pltpu.VMEM((_NB, _T, _D), jnp.float32),
        pltpu.SemaphoreType.DMA,
        pltpu.SemaphoreType.DMA,
    ],
)
def _gather_bcast(idx_hbm, table_hbm, out_hbm, idx_v, rows_v, gsem, wsem):
    wid = lax.axis_index("s") * _NC + lax.axis_index("c")
    base = wid * _NB

    pltpu.sync_copy(idx_hbm.at[wid], idx_v)

    gathers = [
        pltpu.async_copy(table_hbm.at[idx_v.at[i]], rows_v.at[i], gsem)
        for i in range(_NB)
    ]
    for g in gathers:
        g.wait()

    writes = [
        pltpu.async_copy(rows_v, out_hbm.at[pl.ds(base, _NB), j], wsem)
        for j in range(_DEC)
    ]
    for w in writes:
        w.wait()


def kernel(inputs, decoder_length, table):
    del decoder_length
    idx = inputs.reshape(_NW, _NB, _T)
    out = _gather_bcast(idx, table)
    return out.reshape(_B, _DEC, _T * _D)


# speedup vs baseline: 1.2635x; 1.2635x over previous
import functools

import jax
import jax.numpy as jnp
from jax import lax
from jax.experimental import pallas as pl
from jax.experimental.pallas import tpu as pltpu
from jax.experimental.pallas import tpu_sc as plsc

_B = 1024
_T = 50
_D = 32
_DEC = 20
_VOCAB = 100000
_GW = 128 // _D
_VG = _VOCAB // _GW
_TP = 64
_TG = 56

_info = plsc.get_sparse_core_info()
_NC = _info.num_cores
_NS = _info.num_subcores
_NW = _NC * _NS
_NB = _B // _NW


@functools.partial(
    pl.kernel,
    mesh=plsc.VectorSubcoreMesh(core_axis_name="c", subcore_axis_name="s"),
    out_type=jax.ShapeDtypeStruct((_B, _DEC, _T * _D), jnp.float32),
    scratch_types=[
        pltpu.VMEM((_NB * _TP,), jnp.int32),
        pltpu.VMEM((_NB * _TP,), jnp.int32),
        pltpu.VMEM((_TG, 128), jnp.float32),
        pltpu.VMEM((_NB, _T * _D), jnp.float32),
        pltpu.SemaphoreType.DMA,
        pltpu.SemaphoreType.DMA,
    ],
)
def _gather_bcast(idx_hbm, tbl_hbm, out_hbm, idx_v, gidx_v, gbuf, rows_v,
                  gsem, wsem):
    wid = lax.axis_index("s") * _NC + lax.axis_index("c")
    base = wid * _NB

    pltpu.sync_copy(idx_hbm.at[pl.ds(base * _TP, _NB * _TP)], idx_v)

    def row_body(i, carry):
        for c in range(_TP // 16):
            x = idx_v[pl.ds(i * _TP + c * 16, 16)]
            g = jnp.minimum(lax.shift_right_logical(x, 2), _VG - 1)
            gidx_v[pl.ds(i * _TP + c * 16, 16)] = g

        pltpu.async_copy(tbl_hbm.at[gidx_v.at[pl.ds(i * _TP, _TG)]],
                         gbuf, gsem).wait()

        for c in range(-(-_T // 16)):
            x = idx_v[pl.ds(i * _TP + c * 16, 16)]
            sv = (x & (_GW - 1)) * _D
            for l in range(min(16, _T - c * 16)):
                t = c * 16 + l
                s = pl.multiple_of(sv[l], _D)
                for h in range(_D // 16):
                    v = gbuf[t, pl.ds(s + h * 16, 16)]
                    rows_v[i, pl.ds(t * _D + h * 16, 16)] = v
        return carry

    lax.fori_loop(0, _NB, row_body, 0)

    writes = [
        pltpu.async_copy(rows_v, out_hbm.at[pl.ds(base, _NB), j], wsem)
        for j in range(_DEC)
    ]
    for w in writes:
        w.wait()


def kernel(inputs, decoder_length, table):
    del decoder_length
    idx = jnp.pad(inputs, ((0, 0), (0, _TP - _T))).reshape(_B * _TP)
    tbl = table.reshape(_VG, 128)
    return _gather_bcast(idx, tbl)


# speedup vs baseline: 2.3638x; 1.8708x over previous
import functools

import jax
import jax.numpy as jnp
from jax import lax
from jax.experimental import pallas as pl
from jax.experimental.pallas import tpu as pltpu
from jax.experimental.pallas import tpu_sc as plsc

_B = 1024
_T = 50
_D = 32
_DEC = 20

_info = plsc.get_sparse_core_info()
_NC = _info.num_cores
_NS = _info.num_subcores
_NW = _NC * _NS
_NB = _B // _NW

_BB = 16


@functools.partial(
    pl.kernel,
    mesh=plsc.VectorSubcoreMesh(core_axis_name="c", subcore_axis_name="s"),
    out_type=jax.ShapeDtypeStruct((_B * _T, _D), jnp.float32),
    compiler_params=pltpu.CompilerParams(use_tc_tiling_on_sc=False),
    scratch_types=[
        pltpu.VMEM((_NB, _T), jnp.int32),
        pltpu.VMEM((_NB * _T, _D), jnp.float32),
        pltpu.SemaphoreType.DMA,
    ],
)
def _sc_gather(idx_hbm, table_hbm, out_hbm, idx_v, rows_v, gsem):
    wid = lax.axis_index("s") * _NC + lax.axis_index("c")
    base = wid * _NB

    pltpu.sync_copy(idx_hbm.at[pl.ds(base, _NB)], idx_v)

    gathers = [
        pltpu.async_copy(table_hbm.at[idx_v.at[i]],
                         rows_v.at[pl.ds(i * _T, _T)], gsem)
        for i in range(_NB)
    ]
    for g in gathers:
        g.wait()

    pltpu.sync_copy(rows_v, out_hbm.at[pl.ds(base * _T, _NB * _T)])


def _tc_bcast_body(in_ref, out_ref):
    x = in_ref[...]
    out_ref[...] = jnp.broadcast_to(x[:, None, :], (_BB, _DEC, _T * _D))


_tc_bcast = pl.pallas_call(
    _tc_bcast_body,
    grid=(_B // _BB,),
    in_specs=[pl.BlockSpec((_BB, _T * _D), lambda b: (b, 0))],
    out_specs=pl.BlockSpec((_BB, _DEC, _T * _D), lambda b: (b, 0, 0)),
    out_shape=jax.ShapeDtypeStruct((_B, _DEC, _T * _D), jnp.float32),
)


def kernel(inputs, decoder_length, table):
    del decoder_length
    rows = _sc_gather(inputs, table)
    return _tc_bcast(rows.reshape(_B, _T * _D))


# speedup vs baseline: 4.8371x; 2.0463x over previous
import functools

import jax
import jax.numpy as jnp
from jax import lax
from jax.experimental import pallas as pl
from jax.experimental.pallas import tpu as pltpu
from jax.experimental.pallas import tpu_sc as plsc

_B = 1024
_T = 50
_D = 32
_DEC = 20
_F = _T * _D

_info = plsc.get_sparse_core_info()
_NC = _info.num_cores
_NS = _info.num_subcores
_NW = _NC * _NS
_NB = _B // _NW


@functools.partial(
    pl.kernel,
    mesh=plsc.VectorSubcoreMesh(core_axis_name="c", subcore_axis_name="s"),
    out_type=jax.ShapeDtypeStruct((_B * _T, _D), jnp.float32),
    compiler_params=pltpu.CompilerParams(use_tc_tiling_on_sc=False),
    scratch_types=[
        pltpu.VMEM((_NB, _T), jnp.int32),
        pltpu.VMEM((_NB * _T, _D), jnp.float32),
        pltpu.SemaphoreType.DMA,
    ],
)
def _sc_gather(idx_hbm, table_hbm, out_hbm, idx_v, rows_v, gsem):
    wid = lax.axis_index("s") * _NC + lax.axis_index("c")
    base = wid * _NB

    pltpu.sync_copy(idx_hbm.at[pl.ds(base, _NB)], idx_v)

    gathers = [
        pltpu.async_copy(table_hbm.at[idx_v.at[i]],
                         rows_v.at[pl.ds(i * _T, _T)], gsem)
        for i in range(_NB)
    ]
    for g in gathers:
        g.wait()

    pltpu.sync_copy(rows_v, out_hbm.at[pl.ds(base * _T, _NB * _T)])


@functools.partial(
    pl.kernel,
    mesh=plsc.VectorSubcoreMesh(core_axis_name="c", subcore_axis_name="s"),
    out_type=jax.ShapeDtypeStruct((_DEC, _F, _B), jnp.float32),
    scratch_types=[
        pltpu.VMEM((56, _B), jnp.float32),
        pltpu.SemaphoreType.DMA,
    ],
)
def _sc_bcast(gt_hbm, out_hbm, chunk_v, wsem):
    wid = lax.axis_index("s") * _NC + lax.axis_index("c")

    is_big = wid < 8
    off_big = wid * 56
    off_small = 8 * 56 + (wid - 8) * 48
    f0 = jnp.where(is_big, off_big, off_small)
    f0 = pl.multiple_of(f0, 8)

    @pl.when(is_big)
    def _():
        pltpu.sync_copy(gt_hbm.at[pl.ds(f0, 56)], chunk_v)
        writes = [
            pltpu.async_copy(chunk_v, out_hbm.at[j, pl.ds(f0, 56)], wsem)
            for j in range(_DEC)
        ]
        for w in writes:
            w.wait()

    @pl.when(jnp.logical_not(is_big))
    def _():
        pltpu.sync_copy(gt_hbm.at[pl.ds(f0, 48)], chunk_v.at[pl.ds(0, 48)])
        writes = [
            pltpu.async_copy(chunk_v.at[pl.ds(0, 48)],
                             out_hbm.at[j, pl.ds(f0, 48)], wsem)
            for j in range(_DEC)
        ]
        for w in writes:
            w.wait()


def kernel(inputs, decoder_length, table):
    del decoder_length
    rows = _sc_gather(inputs, table)
    gt = rows.reshape(_B, _F).T
    out = _sc_bcast(gt)
    return out.transpose(2, 0, 1)
